# Initial kernel scaffold; baseline (speedup 1.0000x reference)
import functools

import jax
import jax.numpy as jnp
from jax import lax
from jax.experimental import pallas as pl
from jax.experimental.pallas import tpu as pltpu
from jax.experimental.pallas import tpu_sc as plsc

AXES_LIMIT = 40.0
RESOLUTION = 0.25

NC = 2
NS = 16
L = 16
NW = NC * NS

B = 4
C = 128
HW = 320 * 320
N = 20000
NQ = B * N
NP = 81920
BPW = NP // NW
CH = 128
NCHUNK = BPW // CH


def _transpose_tc(fmap3):
    b, c, hw = fmap3.shape
    T = 512
    nblk = hw // T

    def body(in_ref, out_ref):
        out_ref[...] = in_ref[0].T

    return pl.pallas_call(
        body,
        grid=(b, nblk),
        in_specs=[pl.BlockSpec((1, c, T), lambda i, j: (i, 0, j))],
        out_specs=pl.BlockSpec((T, c), lambda i, j: (i * nblk + j, 0)),
        out_shape=jax.ShapeDtypeStruct((b * hw, c), jnp.float32),
    )(fmap3)


_mesh = plsc.VectorSubcoreMesh(
    core_axis_name="c", subcore_axis_name="s", num_cores=NC, num_subcores=NS
)


@functools.partial(
    pl.kernel,
    mesh=_mesh,
    out_type=jax.ShapeDtypeStruct((NP, C), jnp.float32),
    scratch_types=[
        pltpu.VMEM((BPW, 2), jnp.float32),
        pltpu.VMEM((BPW,), jnp.int32),
        pltpu.VMEM((CH, C), jnp.float32),
        pltpu.SemaphoreType.DMA,
    ],
)
def _gather_sc(table_hbm, loc_hbm, out_hbm, loc_v, idx_v, rows_v, sem):
    wid = lax.axis_index("s") * NC + lax.axis_index("c")
    base = wid * BPW

    pltpu.sync_copy(loc_hbm.at[pl.ds(base, BPW), :], loc_v)

    iota = lax.iota(jnp.int32, L)
    zeros = jnp.zeros((L,), jnp.int32)
    ones = jnp.ones((L,), jnp.int32)

    def compute_idx(i, carry):
        rid = i * L + iota
        x = plsc.load_gather(loc_v, [rid, zeros])
        y = plsc.load_gather(loc_v, [rid, ones])
        x = jnp.clip(x, -0.999, 0.999) * AXES_LIMIT
        y = jnp.clip(y, -0.999, 0.999) * AXES_LIMIT
        row = ((AXES_LIMIT - y) / RESOLUTION).astype(jnp.int32)
        col = ((AXES_LIMIT + x) / RESOLUTION).astype(jnp.int32)
        g = base + rid
        bb = jnp.minimum(g // N, B - 1)
        gi = jnp.minimum(bb * HW + row * 320 + col, B * HW - 1)
        idx_v[pl.ds(i * L, L)] = gi
        return carry

    lax.fori_loop(0, BPW // L, compute_idx, 0)

    def gather_chunk(k, carry):
        pltpu.async_copy(
            table_hbm.at[idx_v.at[pl.ds(k * CH, CH)]], rows_v, sem
        ).wait()
        pltpu.sync_copy(rows_v, out_hbm.at[pl.ds(base + k * CH, CH), :])
        return carry

    lax.fori_loop(0, NCHUNK, gather_chunk, 0)


def kernel(fmap, loc):
    b, c, h, w = fmap.shape
    table = _transpose_tc(fmap.reshape(b, c, h * w))
    locp = jnp.pad(loc.reshape(b * N, 2), ((0, NP - b * N), (0, 0)))
    out = _gather_sc(table, locp)
    return out[: b * N].reshape(b, N, c)


# speedup vs baseline: 1.5564x; 1.5564x over previous
import functools

import jax
import jax.numpy as jnp
from jax import lax
from jax.experimental import pallas as pl
from jax.experimental.pallas import tpu as pltpu
from jax.experimental.pallas import tpu_sc as plsc

AXES_LIMIT = 40.0
RESOLUTION = 0.25

NC = 2
NS = 16
NW = NC * NS

B = 4
C = 128
HW = 320 * 320
N = 20000
NQ = B * N
NP = 81920
BPW = NP // NW
CH = 128
NCHUNK = BPW // CH


def _transpose_tc(fmap3):
    b, c, hw = fmap3.shape
    T = 512
    nblk = hw // T

    def body(in_ref, out_ref):
        out_ref[...] = in_ref[0].T

    return pl.pallas_call(
        body,
        grid=(b, nblk),
        in_specs=[pl.BlockSpec((1, c, T), lambda i, j: (i, 0, j))],
        out_specs=pl.BlockSpec((T, c), lambda i, j: (i * nblk + j, 0)),
        out_shape=jax.ShapeDtypeStruct((b * hw, c), jnp.float32),
    )(fmap3)


def _index_tc(xs2, ys2):
    def body(x_ref, y_ref, o_ref):
        x = jnp.clip(x_ref[...], -0.999, 0.999) * AXES_LIMIT
        y = jnp.clip(y_ref[...], -0.999, 0.999) * AXES_LIMIT
        row = ((AXES_LIMIT - y) / RESOLUTION).astype(jnp.int32)
        col = ((AXES_LIMIT + x) / RESOLUTION).astype(jnp.int32)
        r = x.shape[0]
        pos = (
            lax.broadcasted_iota(jnp.int32, (r, 128), 0) * 128
            + lax.broadcasted_iota(jnp.int32, (r, 128), 1)
        )
        bb = jnp.minimum(pos // N, B - 1)
        o_ref[...] = jnp.minimum(bb * HW + row * 320 + col, B * HW - 1)

    return pl.pallas_call(
        body,
        out_shape=jax.ShapeDtypeStruct(xs2.shape, jnp.int32),
    )(xs2, ys2)


_mesh = plsc.VectorSubcoreMesh(
    core_axis_name="c", subcore_axis_name="s", num_cores=NC, num_subcores=NS
)


@functools.partial(
    pl.kernel,
    mesh=_mesh,
    out_type=jax.ShapeDtypeStruct((NP, C), jnp.float32),
    scratch_types=[
        pltpu.VMEM((BPW,), jnp.int32),
        pltpu.VMEM((CH, C), jnp.float32),
        pltpu.SemaphoreType.DMA,
    ],
)
def _gather_sc(table_hbm, idx_hbm, out_hbm, idx_v, rows_v, sem):
    wid = lax.axis_index("s") * NC + lax.axis_index("c")
    base = wid * BPW

    pltpu.sync_copy(idx_hbm.at[pl.ds(base, BPW)], idx_v)

    for k in range(NCHUNK):
        pltpu.async_copy(
            table_hbm.at[idx_v.at[pl.ds(k * CH, CH)]], rows_v, sem
        ).wait()
        pltpu.sync_copy(rows_v, out_hbm.at[pl.ds(base + k * CH, CH), :])


def kernel(fmap, loc):
    b, c, h, w = fmap.shape
    table = _transpose_tc(fmap.reshape(b, c, h * w))
    xs = jnp.pad(loc[..., 0].reshape(b * N), (0, NP - b * N)).reshape(NP // 128, 128)
    ys = jnp.pad(loc[..., 1].reshape(b * N), (0, NP - b * N)).reshape(NP // 128, 128)
    idx = _index_tc(xs, ys).reshape(NP)
    out = _gather_sc(table, idx)
    return out[: b * N].reshape(b, N, c)


# speedup vs baseline: 1.5870x; 1.0197x over previous
import functools

import jax
import jax.numpy as jnp
from jax import lax
from jax.experimental import pallas as pl
from jax.experimental.pallas import tpu as pltpu
from jax.experimental.pallas import tpu_sc as plsc

AXES_LIMIT = 40.0
RESOLUTION = 0.25

NC = 2
NS = 16
NW = NC * NS

B = 4
C = 128
HW = 320 * 320
N = 20000
NQ = B * N
NP = 81920
BPW = NP // NW
CH = 128
NCHUNK = BPW // CH
NBUF = 4


def _transpose_tc(fmap3):
    b, c, hw = fmap3.shape
    T = 512
    nblk = hw // T

    def body(in_ref, out_ref):
        out_ref[...] = in_ref[0].T

    return pl.pallas_call(
        body,
        grid=(b, nblk),
        in_specs=[pl.BlockSpec((1, c, T), lambda i, j: (i, 0, j))],
        out_specs=pl.BlockSpec((T, c), lambda i, j: (i * nblk + j, 0)),
        out_shape=jax.ShapeDtypeStruct((b * hw, c), jnp.float32),
    )(fmap3)


def _index_tc(xs2, ys2):
    def body(x_ref, y_ref, o_ref):
        x = jnp.clip(x_ref[...], -0.999, 0.999) * AXES_LIMIT
        y = jnp.clip(y_ref[...], -0.999, 0.999) * AXES_LIMIT
        row = ((AXES_LIMIT - y) / RESOLUTION).astype(jnp.int32)
        col = ((AXES_LIMIT + x) / RESOLUTION).astype(jnp.int32)
        r = x.shape[0]
        pos = (
            lax.broadcasted_iota(jnp.int32, (r, 128), 0) * 128
            + lax.broadcasted_iota(jnp.int32, (r, 128), 1)
        )
        bb = jnp.minimum(pos // N, B - 1)
        o_ref[...] = jnp.minimum(bb * HW + row * 320 + col, B * HW - 1)

    return pl.pallas_call(
        body,
        out_shape=jax.ShapeDtypeStruct(xs2.shape, jnp.int32),
    )(xs2, ys2)


_mesh = plsc.VectorSubcoreMesh(
    core_axis_name="c", subcore_axis_name="s", num_cores=NC, num_subcores=NS
)


@functools.partial(
    pl.kernel,
    mesh=_mesh,
    out_type=jax.ShapeDtypeStruct((NP, C), jnp.float32),
    scratch_types=[
        pltpu.VMEM((BPW,), jnp.int32),
        pltpu.VMEM((NBUF, CH, C), jnp.float32),
        [pltpu.SemaphoreType.DMA] * NBUF,
        [pltpu.SemaphoreType.DMA] * NBUF,
    ],
)
def _gather_sc(table_hbm, idx_hbm, out_hbm, idx_v, rows_v, gsem, wsem):
    wid = lax.axis_index("s") * NC + lax.axis_index("c")
    base = wid * BPW

    pltpu.sync_copy(idx_hbm.at[pl.ds(base, BPW)], idx_v)

    gcp = {}
    wcp = {}
    for k in range(NCHUNK + 1):
        if k < NCHUNK:
            bi = k % NBUF
            if k >= NBUF:
                wcp[k - NBUF].wait()
            gcp[k] = pltpu.async_copy(
                table_hbm.at[idx_v.at[pl.ds(k * CH, CH)]], rows_v.at[bi], gsem[bi]
            )
        if k >= 1:
            j = k - 1
            bj = j % NBUF
            gcp[j].wait()
            wcp[j] = pltpu.async_copy(
                rows_v.at[bj], out_hbm.at[pl.ds(base + j * CH, CH), :], wsem[bj]
            )
    for j in range(max(0, NCHUNK - NBUF), NCHUNK):
        wcp[j].wait()


def kernel(fmap, loc):
    b, c, h, w = fmap.shape
    table = _transpose_tc(fmap.reshape(b, c, h * w))
    xs = jnp.pad(loc[..., 0].reshape(b * N), (0, NP - b * N)).reshape(NP // 128, 128)
    ys = jnp.pad(loc[..., 1].reshape(b * N), (0, NP - b * N)).reshape(NP // 128, 128)
    idx = _index_tc(xs, ys).reshape(NP)
    out = _gather_sc(table, idx)
    return out[: b * N].reshape(b, N, c)


# speedup vs baseline: 3.1907x; 2.0105x over previous
import functools

import jax
import jax.numpy as jnp
from jax import lax
from jax.experimental import pallas as pl
from jax.experimental.pallas import tpu as pltpu
from jax.experimental.pallas import tpu_sc as plsc

AXES_LIMIT = 40.0
RESOLUTION = 0.25

NC = 2
NS = 16
NW = NC * NS

B = 4
C = 128
HW = 320 * 320
N = 20000
NQ = B * N
CH = 100
NCH = NQ // CH
JPW = NCH // NW
NBUF = 4


def _transpose_tc(fmap3):
    b, c, hw = fmap3.shape
    T = 20480
    nblk = hw // T

    def body(in_ref, out_ref):
        out_ref[...] = in_ref[0].T

    return pl.pallas_call(
        body,
        grid=(b, nblk),
        in_specs=[pl.BlockSpec((1, c, T), lambda i, j: (i, 0, j))],
        out_specs=pl.BlockSpec((T, c), lambda i, j: (i * nblk + j, 0)),
        out_shape=jax.ShapeDtypeStruct((b * hw, c), jnp.float32),
    )(fmap3)


def _index_tc(xs2, ys2):
    def body(x_ref, y_ref, o_ref):
        x = jnp.clip(x_ref[...], -0.999, 0.999) * AXES_LIMIT
        y = jnp.clip(y_ref[...], -0.999, 0.999) * AXES_LIMIT
        row = ((AXES_LIMIT - y) / RESOLUTION).astype(jnp.int32)
        col = ((AXES_LIMIT + x) / RESOLUTION).astype(jnp.int32)
        r = x.shape[0]
        pos = (
            lax.broadcasted_iota(jnp.int32, (r, 128), 0) * 128
            + lax.broadcasted_iota(jnp.int32, (r, 128), 1)
        )
        bb = pos // N
        o_ref[...] = bb * HW + row * 320 + col

    return pl.pallas_call(
        body,
        out_shape=jax.ShapeDtypeStruct(xs2.shape, jnp.int32),
    )(xs2, ys2)


_mesh = plsc.VectorSubcoreMesh(
    core_axis_name="c", subcore_axis_name="s", num_cores=NC, num_subcores=NS
)


@functools.partial(
    pl.kernel,
    mesh=_mesh,
    out_type=jax.ShapeDtypeStruct((NCH, CH, C), jnp.float32),
    scratch_types=[
        pltpu.VMEM((NBUF, 1, CH), jnp.int32),
        pltpu.VMEM((NBUF, CH, C), jnp.float32),
        [pltpu.SemaphoreType.DMA] * NBUF,
        [pltpu.SemaphoreType.DMA] * NBUF,
        [pltpu.SemaphoreType.DMA] * NBUF,
    ],
)
def _gather_sc(table_hbm, idx_hbm, out_hbm, idxc_v, rows_v, isem, gsem, wsem):
    wid = lax.axis_index("s") * NC + lax.axis_index("c")
    base = wid * JPW

    icp = {}
    gcp = {}
    wcp = {}
    for t in range(JPW + 2):
        if t < JPW:
            j = t
            bi = j % NBUF
            if j >= NBUF:
                wcp[j - NBUF].wait()
            icp[j] = pltpu.async_copy(
                idx_hbm.at[base + j], idxc_v.at[bi], isem[bi]
            )
        if 1 <= t <= JPW:
            j = t - 1
            bi = j % NBUF
            icp[j].wait()
            gcp[j] = pltpu.async_copy(
                table_hbm.at[idxc_v.at[bi, 0]], rows_v.at[bi], gsem[bi]
            )
        if 2 <= t:
            j = t - 2
            bi = j % NBUF
            gcp[j].wait()
            wcp[j] = pltpu.async_copy(
                rows_v.at[bi], out_hbm.at[base + j], wsem[bi]
            )
    for j in range(JPW - NBUF, JPW):
        wcp[j].wait()


def kernel(fmap, loc):
    b, c, h, w = fmap.shape
    table = _transpose_tc(fmap.reshape(b, c, h * w))
    xs = loc[..., 0].reshape(NQ // 128, 128)
    ys = loc[..., 1].reshape(NQ // 128, 128)
    idx = _index_tc(xs, ys).reshape(NCH, 1, CH)
    out = _gather_sc(table, idx)
    return out.reshape(b, N, c)
